# baseline (speedup 1.0000x reference)
import functools

import jax
import jax.numpy as jnp
from jax import lax
from jax.experimental import pallas as pl
from jax.experimental.pallas import tpu as pltpu
from jax.experimental.pallas import tpu_sc as plsc

_D = 64
_N = 16384
_C = 20
_VA = 1000000
_VB = 100000
_VC = 1000
_NW = 32
_NBLK = _N // 128
_BPW = _NBLK // _NW


def _make_kernel():
    mesh = plsc.VectorSubcoreMesh(core_axis_name="c", subcore_axis_name="s")
    out_t = [jax.ShapeDtypeStruct((_C, 8, _NBLK, 8, 128), jnp.float32)
             for _ in range(3)]
    scratch = [
        pltpu.VMEM((_C, 128), jnp.int32),
        pltpu.VMEM((_C, 128), jnp.int32),
        pltpu.VMEM((_C, 128), jnp.int32),
        pltpu.VMEM((128, 128), jnp.float32),
        pltpu.VMEM((128, 128), jnp.float32),
        pltpu.VMEM((8, 8, 128), jnp.float32),
        pltpu.SemaphoreType.DMA,
    ]

    @functools.partial(
        pl.kernel, out_type=out_t, mesh=mesh, scratch_types=scratch,
        compiler_params=pltpu.CompilerParams(
            use_tc_tiling_on_sc=False, needs_layout_passes=False))
    def k(ta, tb, tc_, wa, wb, wc, oa, ob, oc,
          idxblk, jblk, hblk, pairs0, pairs1, tbuf, gsem):
        wid = lax.axis_index("s") * 2 + lax.axis_index("c")
        pairs = (pairs0, pairs1)
        iota16 = lax.iota(jnp.int32, 16)

        for cat, wp, out in ((ta, wa, oa), (tb, wb, ob), (tc_, wc, oc)):
            def g_fire(c, buf, wp=wp):
                pltpu.async_copy(wp.at[jblk.at[c]], buf, gsem)

            def g_wait(buf, wp=wp):
                pltpu.make_async_copy(wp.at[jblk.at[0]], buf, gsem).wait()

            def transpose_c(c, buf, out=out):
                def dt_body(dt, _, buf=buf):
                    for g in range(8):
                        row16 = iota16 + (g * 16)
                        h16 = hblk[c, pl.ds(g * 16, 16)]
                        for dr in range(8):
                            col16 = h16 + (dt * 8 + dr)
                            v16 = plsc.load_gather(buf, [row16, col16])
                            tbuf[dt, dr, pl.ds(g * 16, 16)] = v16
                    return 0
                lax.fori_loop(0, 8, dt_body, 0)

            def blk_body(b, _, cat=cat, out=out):
                nt = wid * _BPW + b
                n0 = nt * 128
                pltpu.sync_copy(cat.at[pl.ds(0, _C), pl.ds(n0, 128)], idxblk)

                def jh_body(ci, _):
                    for g in range(8):
                        v = idxblk[ci, pl.ds(g * 16, 16)]
                        jblk[ci, pl.ds(g * 16, 16)] = v >> 1
                        hblk[ci, pl.ds(g * 16, 16)] = (v & 1) << 6
                    return 0
                lax.fori_loop(0, _C, jh_body, 0)

                g_fire(0, pairs0)

                def c_body(cc, _, out=out):
                    for par in range(2):
                        c = cc * 2 + par
                        g_wait(pairs[par])

                        @pl.when(c < _C - 1)
                        def _(par=par, c=c):
                            g_fire(c + 1, pairs[1 - par])

                        transpose_c(c, pairs[par])
                        pltpu.sync_copy(tbuf, out.at[c, pl.ds(0, 8), nt])
                    return 0
                lax.fori_loop(0, _C // 2, c_body, 0)
                return 0

            lax.fori_loop(0, _BPW, blk_body, 0)

    return k


_gather3 = _make_kernel()


def kernel(cat_a, cat_b, cat_c, W_cat_a, W_cat_b, W_cat_c):
    wpa = W_cat_a.reshape(_VA // 2, 128)
    wpb = W_cat_b.reshape(_VB // 2, 128)
    wpc = W_cat_c.reshape(_VC // 2, 128)
    oa, ob, oc = _gather3(cat_a.T, cat_b.T, cat_c.T, wpa, wpb, wpc)

    def fix(o):
        return o.transpose(2, 4, 0, 1, 3).reshape(_N, _C, _D)

    return (fix(oa), fix(ob), fix(oc))


# speedup vs baseline: 1.6664x; 1.6664x over previous
import functools

import jax
import jax.numpy as jnp
from jax import lax
from jax.experimental import pallas as pl
from jax.experimental.pallas import tpu as pltpu
from jax.experimental.pallas import tpu_sc as plsc

_D = 64
_N = 16384
_C = 20
_TOT = _N * _C
_NW = 32
_PER_W = _TOT // _NW
_CH = 512
_NCH = _PER_W // _CH
_NB = 2
_NGRP = _NCH // _NB


def _make_kernel(vocab):
    mesh = plsc.VectorSubcoreMesh(core_axis_name="c", subcore_axis_name="s")
    out_t = jax.ShapeDtypeStruct((_TOT, _D), jnp.float32)
    scratch = [
        pltpu.VMEM((_PER_W,), jnp.int32),
        pltpu.VMEM((_CH, _D), jnp.float32),
        pltpu.VMEM((_CH, _D), jnp.float32),
        pltpu.SemaphoreType.DMA,
        pltpu.SemaphoreType.DMA,
    ]

    @functools.partial(
        pl.kernel, out_type=out_t, mesh=mesh, scratch_types=scratch,
        compiler_params=pltpu.CompilerParams(use_tc_tiling_on_sc=False))
    def k(idx_hbm, tab_hbm, out_hbm, idxall, rows0, rows1, gsem, wsem):
        wid = lax.axis_index("s") * 2 + lax.axis_index("c")
        base = wid * _PER_W
        rows = (rows0, rows1)

        def g_fire(j, b):
            pltpu.async_copy(
                tab_hbm.at[idxall.at[pl.ds(j * _CH, _CH)]], rows[b], gsem)

        def g_wait(b):
            pltpu.make_async_copy(
                tab_hbm.at[idxall.at[pl.ds(0, _CH)]], rows[b], gsem).wait()

        def w_fire(j, b):
            pltpu.async_copy(
                rows[b], out_hbm.at[pl.ds(base + j * _CH, _CH)], wsem)

        def w_wait(b):
            pltpu.make_async_copy(
                rows[b], out_hbm.at[pl.ds(base, _CH)], wsem).wait()

        pltpu.sync_copy(idx_hbm.at[pl.ds(base, _PER_W)], idxall)
        g_fire(0, 0)
        g_fire(1, 1)
        g_wait(0)
        w_fire(0, 0)
        w_wait(0)
        g_fire(2, 0)
        g_wait(1)
        w_fire(1, 1)

        def grp(g, _):
            for b in range(_NB):
                j = g * _NB + b
                w_wait((b + 1) % _NB)
                g_fire(j + 1, (b + 1) % _NB)
                g_wait(b)
                w_fire(j, b)
            return 0

        lax.fori_loop(1, _NGRP - 1, grp, 0)

        w_wait(1)
        g_fire(_NCH - 1, 1)
        g_wait(0)
        w_fire(_NCH - 2, 0)
        w_wait(0)
        g_wait(1)
        w_fire(_NCH - 1, 1)
        w_wait(1)

    return k


_KERNELS = {v: _make_kernel(v) for v in (1000000, 100000, 1000)}


def kernel(cat_a, cat_b, cat_c, W_cat_a, W_cat_b, W_cat_c):
    shape = (_N, _C, _D)
    ob = _KERNELS[100000](cat_b.reshape(-1), W_cat_b).reshape(shape)
    oc = _KERNELS[1000](cat_c.reshape(-1), W_cat_c).reshape(shape)
    oa = _KERNELS[1000000](cat_a.reshape(-1), W_cat_a).reshape(shape)
    return (oa, ob, oc)


# speedup vs baseline: 1.6800x; 1.0082x over previous
import functools

import jax
import jax.numpy as jnp
from jax import lax
from jax.experimental import pallas as pl
from jax.experimental.pallas import tpu as pltpu
from jax.experimental.pallas import tpu_sc as plsc

_D = 64
_N = 16384
_C = 20
_NW = 32
_PW = _N // _NW
_NB = 2
_NGRP = _C // _NB


def _make_kernel(vocab):
    mesh = plsc.VectorSubcoreMesh(core_axis_name="c", subcore_axis_name="s")
    out_t = jax.ShapeDtypeStruct((_N, _C, _D), jnp.float32)
    scratch = [
        pltpu.VMEM((_C, _PW), jnp.int32),
        pltpu.VMEM((_PW, _D), jnp.float32),
        pltpu.VMEM((_PW, _D), jnp.float32),
        pltpu.SemaphoreType.DMA,
        pltpu.SemaphoreType.DMA,
    ]

    @functools.partial(
        pl.kernel, out_type=out_t, mesh=mesh, scratch_types=scratch,
        compiler_params=pltpu.CompilerParams(use_tc_tiling_on_sc=False))
    def k(idx_hbm, tab_hbm, out_hbm, idxall, rows0, rows1, gsem, wsem):
        wid = lax.axis_index("s") * 2 + lax.axis_index("c")
        n0 = wid * _PW
        rows = (rows0, rows1)

        def g_fire(c, b):
            pltpu.async_copy(tab_hbm.at[idxall.at[c]], rows[b], gsem)

        def g_wait(b):
            pltpu.make_async_copy(
                tab_hbm.at[idxall.at[0]], rows[b], gsem).wait()

        def w_fire(c, b):
            pltpu.async_copy(rows[b], out_hbm.at[pl.ds(n0, _PW), c], wsem)

        def w_wait(b):
            pltpu.make_async_copy(
                rows[b], out_hbm.at[pl.ds(n0, _PW), 0], wsem).wait()

        pltpu.sync_copy(idx_hbm.at[pl.ds(0, _C), pl.ds(n0, _PW)], idxall)
        g_fire(0, 0)
        g_fire(1, 1)
        g_wait(0)
        w_fire(0, 0)
        w_wait(0)
        g_fire(2, 0)
        g_wait(1)
        w_fire(1, 1)

        def grp(g, _):
            for b in range(_NB):
                c = g * _NB + b
                w_wait((b + 1) % _NB)
                g_fire(c + 1, (b + 1) % _NB)
                g_wait(b)
                w_fire(c, b)
            return 0

        lax.fori_loop(1, _NGRP - 1, grp, 0)

        w_wait(1)
        g_fire(_C - 1, 1)
        g_wait(0)
        w_fire(_C - 2, 0)
        w_wait(0)
        g_wait(1)
        w_fire(_C - 1, 1)
        w_wait(1)

    return k


_KERNELS = {v: _make_kernel(v) for v in (1000000, 100000, 1000)}


def kernel(cat_a, cat_b, cat_c, W_cat_a, W_cat_b, W_cat_c):
    ob = _KERNELS[100000](cat_b.T, W_cat_b)
    oc = _KERNELS[1000](cat_c.T, W_cat_c)
    oa = _KERNELS[1000000](cat_a.T, W_cat_a)
    return (oa, ob, oc)
